# baseline (speedup 1.0000x reference)
import jax
import jax.numpy as jnp
from jax.experimental import pallas as pl
from jax.experimental.pallas import tpu as pltpu

_B = 1024


def _body(fs_ref, w_ref, key_ref, kx_ref, z_ref, out_ref, obuf, carry, sc, sem):
    del z_ref
    b = pl.program_id(0)
    B, CIN = fs_ref.shape
    COUT = w_ref.shape[1]
    NTAP = w_ref.shape[0] // CIN

    @pl.when(b == 0)
    def _():
        sc[0] = -1
        sc[1] = -1
        carry[...] = jnp.zeros_like(carry)

    keys = key_ref[0]
    kx = kx_ref[0]
    last_key = sc[0]
    base = sc[1]

    prev = jnp.concatenate(
        [jnp.full((1, 1), last_key, jnp.int32), keys[:, :-1]], axis=1)
    flags = (keys != prev).astype(jnp.float32)
    i0 = jax.lax.broadcasted_iota(jnp.int32, (B, B), 0)
    i1 = jax.lax.broadcasted_iota(jnp.int32, (B, B), 1)
    tri = (i0 <= i1).astype(jnp.float32)
    csum = jnp.dot(flags, tri, preferred_element_type=jnp.float32)
    rf = base.astype(jnp.float32) + csum
    r0f = rf[0, 0]
    loc = rf - r0f

    feats = fs_ref[...]
    taps = (jax.lax.broadcasted_iota(jnp.int32, (B, NTAP), 1)
            == kx.reshape(B, 1)).astype(feats.dtype)
    fexp = (feats[:, None, :] * taps[:, :, None]).reshape(B, NTAP * CIN)
    t = jnp.dot(fexp.astype(jnp.bfloat16), w_ref[...],
                preferred_element_type=jnp.float32)

    locf = loc
    M = (i0.astype(jnp.float32) == locf).astype(jnp.bfloat16)
    out_local = jnp.dot(M, t.astype(jnp.bfloat16),
                        preferred_element_type=jnp.float32)

    cont = keys[0, 0] == last_key
    rowmask = (jax.lax.broadcasted_iota(jnp.int32, (B, 1), 0) == 0) & cont
    out_local = out_local + jnp.where(rowmask, carry[0:1, :], 0.0)

    lB = loc[0, B - 1].astype(jnp.int32)
    selv = (jax.lax.broadcasted_iota(jnp.int32, (1, B), 1) == lB)
    carry[0:1, :] = jnp.dot(selv.astype(jnp.float32), out_local,
                            preferred_element_type=jnp.float32)
    sc[0] = keys[0, B - 1]
    sc[1] = rf[0, B - 1].astype(jnp.int32)

    slot = jax.lax.rem(b, 2)
    r0 = r0f.astype(jnp.int32)
    obuf[slot] = out_local

    @pl.when(b > 0)
    def _():
        prev_r0 = sc[2]
        pltpu.make_async_copy(
            obuf.at[1 - slot], out_ref.at[pl.ds(prev_r0, B), :], sem).wait()

    sc[2] = r0
    cp = pltpu.make_async_copy(obuf.at[slot], out_ref.at[pl.ds(r0, B), :], sem)
    cp.start()

    @pl.when(b == pl.num_programs(0) - 1)
    def _():
        pltpu.make_async_copy(
            obuf.at[slot], out_ref.at[pl.ds(r0, B), :], sem).wait()


def kernel(features, coords, weight):
    N, CIN = features.shape
    NTAP, _, COUT = weight.shape
    half = 256

    oc = coords >> 1
    key = (oc[:, 0] * half + oc[:, 1]) * half + oc[:, 2]
    kidx = (coords[:, 0] & 1) * 4 + (coords[:, 1] & 1) * 2 + (coords[:, 2] & 1)

    perm = jnp.argsort(key)
    skey = key[perm].astype(jnp.int32)
    fs = features[perm]
    kx = kidx[perm].astype(jnp.int32)

    B = _B
    NB = (N + B - 1) // B
    Npad = NB * B
    pad = Npad - N
    if pad:
        fs = jnp.concatenate([fs, jnp.zeros((pad, CIN), fs.dtype)])
        skey = jnp.concatenate(
            [skey, jnp.full((pad,), jnp.int32(2147483647))])
        kx = jnp.concatenate([kx, jnp.zeros((pad,), jnp.int32)])

    keys3 = skey.reshape(NB, 1, B)
    kx3 = kx.reshape(NB, 1, B)
    wflat = weight.reshape(NTAP * CIN, COUT).astype(jnp.bfloat16)
    zout = jnp.zeros((Npad, COUT), jnp.float32)

    out = pl.pallas_call(
        _body,
        grid=(NB,),
        in_specs=[
            pl.BlockSpec((B, CIN), lambda b: (b, 0)),
            pl.BlockSpec((NTAP * CIN, COUT), lambda b: (0, 0)),
            pl.BlockSpec((1, 1, B), lambda b: (b, 0, 0)),
            pl.BlockSpec((1, 1, B), lambda b: (b, 0, 0)),
            pl.BlockSpec(memory_space=pl.ANY),
        ],
        out_specs=pl.BlockSpec(memory_space=pl.ANY),
        out_shape=jax.ShapeDtypeStruct((Npad, COUT), jnp.float32),
        input_output_aliases={4: 0},
        scratch_shapes=[
            pltpu.VMEM((2, B, COUT), jnp.float32),
            pltpu.VMEM((8, COUT), jnp.float32),
            pltpu.SMEM((4,), jnp.int32),
            pltpu.SemaphoreType.DMA,
        ],
        compiler_params=pltpu.CompilerParams(
            dimension_semantics=("arbitrary",)),
    )(fs, wflat, keys3, kx3, zout)
    return out[:N]


# speedup vs baseline: 1.1746x; 1.1746x over previous
import jax
import jax.numpy as jnp
from jax.experimental import pallas as pl
from jax.experimental.pallas import tpu as pltpu

_B = 1024


def _body(fs_ref, w_ref, key_ref, kx_ref, z_ref, out_ref, obuf, carry, sc, sem):
    del z_ref
    b = pl.program_id(0)
    B, CIN = fs_ref.shape
    NTAP = 8
    COUT = w_ref.shape[1] // NTAP

    @pl.when(b == 0)
    def _():
        sc[0] = -1
        sc[1] = -1
        carry[...] = jnp.zeros_like(carry)

    keys = key_ref[0]
    kx = kx_ref[0]
    last_key = sc[0]
    base = sc[1]

    prev = jnp.concatenate(
        [jnp.full((1, 1), last_key, jnp.int32), keys[:, :-1]], axis=1)
    flags = (keys != prev).astype(jnp.float32)
    csum = flags
    d = 1
    while d < B:
        csum = csum + jnp.concatenate(
            [jnp.zeros((1, d), jnp.float32), csum[:, :B - d]], axis=1)
        d *= 2
    rf = base.astype(jnp.float32) + csum
    r0f = rf[0, 0]
    loc = rf - r0f

    feats = fs_ref[...].astype(jnp.bfloat16)
    z = jnp.dot(feats, w_ref[...], preferred_element_type=jnp.float32)
    kxc = kx.reshape(B, 1)
    t = jnp.zeros((B, COUT), jnp.float32)
    for k in range(NTAP):
        t = t + jnp.where(kxc == k, z[:, k * COUT:(k + 1) * COUT], 0.0)

    i0 = jax.lax.broadcasted_iota(jnp.int32, (B, B), 0)
    locf = loc
    M = (i0.astype(jnp.float32) == locf).astype(jnp.bfloat16)
    out_local = jnp.dot(M, t.astype(jnp.bfloat16),
                        preferred_element_type=jnp.float32)

    cont = keys[0, 0] == last_key
    rowmask = (jax.lax.broadcasted_iota(jnp.int32, (B, 1), 0) == 0) & cont
    out_local = out_local + jnp.where(rowmask, carry[0:1, :], 0.0)

    lB = loc[0, B - 1].astype(jnp.int32)
    selv = (jax.lax.broadcasted_iota(jnp.int32, (1, B), 1) == lB)
    carry[0:1, :] = jnp.dot(selv.astype(jnp.float32), out_local,
                            preferred_element_type=jnp.float32)
    sc[0] = keys[0, B - 1]
    sc[1] = rf[0, B - 1].astype(jnp.int32)

    slot = jax.lax.rem(b, 2)
    r0 = r0f.astype(jnp.int32)
    obuf[slot] = out_local

    @pl.when(b > 0)
    def _():
        prev_r0 = sc[2]
        pltpu.make_async_copy(
            obuf.at[1 - slot], out_ref.at[pl.ds(prev_r0, B), :], sem).wait()

    sc[2] = r0
    cp = pltpu.make_async_copy(obuf.at[slot], out_ref.at[pl.ds(r0, B), :], sem)
    cp.start()

    @pl.when(b == pl.num_programs(0) - 1)
    def _():
        pltpu.make_async_copy(
            obuf.at[slot], out_ref.at[pl.ds(r0, B), :], sem).wait()


def kernel(features, coords, weight):
    N, CIN = features.shape
    NTAP, _, COUT = weight.shape
    half = 256

    oc = coords >> 1
    key = (oc[:, 0] * half + oc[:, 1]) * half + oc[:, 2]
    kidx = (coords[:, 0] & 1) * 4 + (coords[:, 1] & 1) * 2 + (coords[:, 2] & 1)

    perm = jnp.argsort(key)
    skey = key[perm].astype(jnp.int32)
    fs = features[perm]
    kx = kidx[perm].astype(jnp.int32)

    B = _B
    NB = (N + B - 1) // B
    Npad = NB * B
    pad = Npad - N
    if pad:
        fs = jnp.concatenate([fs, jnp.zeros((pad, CIN), fs.dtype)])
        skey = jnp.concatenate(
            [skey, jnp.full((pad,), jnp.int32(2147483647))])
        kx = jnp.concatenate([kx, jnp.zeros((pad,), jnp.int32)])

    keys3 = skey.reshape(NB, 1, B)
    kx3 = kx.reshape(NB, 1, B)
    wall = weight.transpose(1, 0, 2).reshape(CIN, NTAP * COUT).astype(
        jnp.bfloat16)
    zout = jnp.zeros((Npad, COUT), jnp.float32)

    out = pl.pallas_call(
        _body,
        grid=(NB,),
        in_specs=[
            pl.BlockSpec((B, CIN), lambda b: (b, 0)),
            pl.BlockSpec((CIN, NTAP * COUT), lambda b: (0, 0)),
            pl.BlockSpec((1, 1, B), lambda b: (b, 0, 0)),
            pl.BlockSpec((1, 1, B), lambda b: (b, 0, 0)),
            pl.BlockSpec(memory_space=pl.ANY),
        ],
        out_specs=pl.BlockSpec(memory_space=pl.ANY),
        out_shape=jax.ShapeDtypeStruct((Npad, COUT), jnp.float32),
        input_output_aliases={4: 0},
        scratch_shapes=[
            pltpu.VMEM((2, B, COUT), jnp.float32),
            pltpu.VMEM((8, COUT), jnp.float32),
            pltpu.SMEM((4,), jnp.int32),
            pltpu.SemaphoreType.DMA,
        ],
        compiler_params=pltpu.CompilerParams(
            dimension_semantics=("arbitrary",)),
    )(fs, wall, keys3, kx3, zout)
    return out[:N]
